# baseline (speedup 1.0000x reference)
import jax
import jax.numpy as jnp
from jax import lax
from jax.experimental import pallas as pl
from jax.experimental.pallas import tpu as pltpu

_TB = 4096


def _make_body(s_dim, a_dim):
    row_bh = s_dim + a_dim
    row_wo = row_bh + 1
    row_bo = row_wo + 1

    def body(s_ref, a_ref, wt_ref, q_ref):
        x = jnp.concatenate(
            [s_ref[...].astype(jnp.bfloat16), a_ref[...].astype(jnp.bfloat16)],
            axis=1)
        wt = wt_ref[:, :row_bh].astype(jnp.bfloat16)
        ht = lax.dot_general(wt, x, (((1,), (1,)), ((), ())),
                             preferred_element_type=jnp.float32)
        ht = jnp.maximum(ht + wt_ref[:, row_bh:row_bh + 1], 0.0)
        q = jnp.sum(ht * wt_ref[:, row_wo:row_wo + 1], axis=0, keepdims=True)
        q_ref[...] = q + wt_ref[0, row_bo]

    return body


def kernel(s, a, w_all):
    B, s_dim = s.shape
    a_dim = a.shape[1]
    k_rows, hidden = w_all.shape

    wt_all = jnp.transpose(w_all)

    tb = _TB if B % _TB == 0 else min(B, 512)
    pad = (-B) % tb
    if pad:
        s = jnp.pad(s, ((0, pad), (0, 0)))
        a = jnp.pad(a, ((0, pad), (0, 0)))
    bp = B + pad

    q = pl.pallas_call(
        _make_body(s_dim, a_dim),
        out_shape=jax.ShapeDtypeStruct((1, bp), jnp.float32),
        grid=(bp // tb,),
        in_specs=[
            pl.BlockSpec((tb, s_dim), lambda i: (i, 0)),
            pl.BlockSpec((tb, a_dim), lambda i: (i, 0)),
            pl.BlockSpec((hidden, k_rows), lambda i: (0, 0)),
        ],
        out_specs=pl.BlockSpec((1, tb), lambda i: (0, i)),
        compiler_params=pltpu.CompilerParams(
            dimension_semantics=("arbitrary",),
            vmem_limit_bytes=64 << 20,
        ),
    )(s, a, wt_all)
    return jnp.reshape(q[:, :B], (B, 1))


# speedup vs baseline: 1.1524x; 1.1524x over previous
import jax
import jax.numpy as jnp
from jax import lax
from jax.experimental import pallas as pl
from jax.experimental.pallas import tpu as pltpu

_TB = 8192


def _make_body(s_dim, a_dim):
    row_bh = s_dim + a_dim
    row_wo = row_bh + 1
    row_bo = row_wo + 1

    def body(s_ref, a_ref, wt_ref, q_ref):
        x = jnp.concatenate(
            [s_ref[...].astype(jnp.bfloat16), a_ref[...].astype(jnp.bfloat16)],
            axis=1)
        wt = wt_ref[:, :row_bh].astype(jnp.bfloat16)
        ht = lax.dot_general(wt, x, (((1,), (1,)), ((), ())),
                             preferred_element_type=jnp.float32)
        ht = jnp.maximum(ht + wt_ref[:, row_bh:row_bh + 1], 0.0)
        q = jnp.sum(ht * wt_ref[:, row_wo:row_wo + 1], axis=0, keepdims=True)
        q_ref[...] = q + wt_ref[0, row_bo]

    return body


def kernel(s, a, w_all):
    B, s_dim = s.shape
    a_dim = a.shape[1]
    k_rows, hidden = w_all.shape

    wt_all = jnp.transpose(w_all)

    tb = _TB if B % _TB == 0 else min(B, 512)
    pad = (-B) % tb
    if pad:
        s = jnp.pad(s, ((0, pad), (0, 0)))
        a = jnp.pad(a, ((0, pad), (0, 0)))
    bp = B + pad

    q = pl.pallas_call(
        _make_body(s_dim, a_dim),
        out_shape=jax.ShapeDtypeStruct((1, bp), jnp.float32),
        grid=(bp // tb,),
        in_specs=[
            pl.BlockSpec((tb, s_dim), lambda i: (i, 0)),
            pl.BlockSpec((tb, a_dim), lambda i: (i, 0)),
            pl.BlockSpec((hidden, k_rows), lambda i: (0, 0)),
        ],
        out_specs=pl.BlockSpec((1, tb), lambda i: (0, i)),
        compiler_params=pltpu.CompilerParams(
            dimension_semantics=("arbitrary",),
            vmem_limit_bytes=64 << 20,
        ),
    )(s, a, wt_all)
    return jnp.reshape(q[:, :B], (B, 1))
